# baseline (speedup 1.0000x reference)
import jax
import jax.numpy as jnp
from jax.experimental import pallas as pl

K = 1024
D = 64
BLK_M = 512


def _vq_kernel(x_ref, cb_ref, zn_ref, en_ref, out_ref):
    x = x_ref[...]
    cb = cb_ref[...]
    znorm = zn_ref[...]
    enorm = en_ref[...]
    scores = jax.lax.dot_general(
        x, cb,
        dimension_numbers=(((1,), (1,)), ((), ())),
        preferred_element_type=jnp.float32,
    )
    dists = (znorm - 2.0 * scores) + enorm
    minval = jnp.min(dists, axis=1, keepdims=True)
    ids = jax.lax.broadcasted_iota(jnp.int32, dists.shape, 1)
    out_ref[...] = jnp.min(jnp.where(dists == minval, ids, K), axis=1)


@jax.jit
def kernel(z_e_x, codebook):
    b, t, d = z_e_x.shape
    m = b * t
    flat = z_e_x.reshape(m, d)
    znorm = jnp.sum(flat ** 2, axis=1, keepdims=True)
    enorm = jnp.sum(codebook ** 2, axis=1)[None, :]
    grid = (m // BLK_M,)
    idx = pl.pallas_call(
        _vq_kernel,
        grid=grid,
        in_specs=[
            pl.BlockSpec((BLK_M, d), lambda i: (i, 0)),
            pl.BlockSpec((K, d), lambda i: (0, 0)),
            pl.BlockSpec((BLK_M, 1), lambda i: (i, 0)),
            pl.BlockSpec((1, K), lambda i: (0, 0)),
        ],
        out_specs=pl.BlockSpec((BLK_M,), lambda i: (i,)),
        out_shape=jax.ShapeDtypeStruct((m,), jnp.int32),
    )(flat, codebook, znorm, enorm)
    return idx.reshape(b, t)


# speedup vs baseline: 1.5516x; 1.5516x over previous
import jax
import jax.numpy as jnp
from jax.experimental import pallas as pl

K = 1024
D = 64
CHUNK = 512
LANES = 128
NSLICE = K // LANES
ROWS_PER_BLOCK = 8
T_FIXED = 1024
CHUNKS_PER_ROW = T_FIXED // CHUNK
CHUNKS_PER_BLOCK = ROWS_PER_BLOCK * CHUNKS_PER_ROW


def _vq_kernel(x_ref, cb2_ref, znl_ref, en_ref, out_ref):
    j = pl.program_id(0) % CHUNKS_PER_BLOCK
    r = j // CHUNKS_PER_ROW
    h = j % CHUNKS_PER_ROW
    x = x_ref[r, pl.ds(h * CHUNK, CHUNK), :]
    cb2 = cb2_ref[...]
    znorm = znl_ref[0].T
    enorm = en_ref[...]
    scores2 = jax.lax.dot_general(
        x, cb2,
        dimension_numbers=(((1,), (1,)), ((), ())),
        preferred_element_type=jnp.float32,
    )
    lane = jax.lax.broadcasted_iota(jnp.int32, (1, LANES), 1).astype(jnp.float32)
    m = (znorm - scores2[:, 0:LANES]) + enorm[:, 0:LANES]
    c = jnp.broadcast_to(lane, (CHUNK, LANES))
    for s in range(1, NSLICE):
        d_s = ((znorm - scores2[:, s * LANES:(s + 1) * LANES])
               + enorm[:, s * LANES:(s + 1) * LANES])
        lt = d_s < m
        m = jnp.where(lt, d_s, m)
        c = jnp.where(lt, lane + float(s * LANES), c)
    mt = m.T
    ct = c.T
    b = jnp.min(mt, axis=0, keepdims=True)
    cand = jnp.where(mt == b, ct, float(K))
    res = jnp.min(cand, axis=0).astype(jnp.int32)
    out_ref[r, pl.ds(h * CHUNK, CHUNK)] = res


@jax.jit
def kernel(z_e_x, codebook):
    b, t, d = z_e_x.shape
    m = b * t
    nchunks = m // CHUNK
    cb2 = 2.0 * codebook
    flat = z_e_x.reshape(m, d)
    znl = jnp.sum(flat ** 2, axis=1).reshape(nchunks, 1, CHUNK)
    enorm = jnp.sum(codebook ** 2, axis=1)[None, :]
    out = pl.pallas_call(
        _vq_kernel,
        grid=(nchunks,),
        in_specs=[
            pl.BlockSpec((ROWS_PER_BLOCK, t, d),
                         lambda i: (i // CHUNKS_PER_BLOCK, 0, 0)),
            pl.BlockSpec((K, d), lambda i: (0, 0)),
            pl.BlockSpec((1, 1, CHUNK), lambda i: (i, 0, 0)),
            pl.BlockSpec((1, K), lambda i: (0, 0)),
        ],
        out_specs=pl.BlockSpec((ROWS_PER_BLOCK, t),
                               lambda i: (i // CHUNKS_PER_BLOCK, 0)),
        out_shape=jax.ShapeDtypeStruct((b, t), jnp.int32),
    )(z_e_x, cb2, znl, enorm)
    return out


# speedup vs baseline: 1.7821x; 1.1485x over previous
import jax
import jax.numpy as jnp
from jax.experimental import pallas as pl

K = 1024
D = 64
CHUNK = 1024
LANES = 128
NSLICE = K // LANES
ROWS_PER_BLOCK = 8
T_FIXED = 1024
CHUNKS_PER_ROW = T_FIXED // CHUNK
CHUNKS_PER_BLOCK = ROWS_PER_BLOCK * CHUNKS_PER_ROW


def _vq_kernel(x_ref, cb2_ref, znl_ref, en_ref, out_ref):
    j = pl.program_id(0) % CHUNKS_PER_BLOCK
    r = j // CHUNKS_PER_ROW
    h = j % CHUNKS_PER_ROW
    x = x_ref[r, pl.ds(h * CHUNK, CHUNK), :]
    cb2 = cb2_ref[...]
    znorm = znl_ref[0].T
    enorm = en_ref[...]
    scores2 = jax.lax.dot_general(
        x, cb2,
        dimension_numbers=(((1,), (1,)), ((), ())),
        preferred_element_type=jnp.float32,
    )
    lane = jax.lax.broadcasted_iota(jnp.int32, (1, LANES), 1).astype(jnp.float32)
    m = (znorm - scores2[:, 0:LANES]) + enorm[:, 0:LANES]
    c = jnp.broadcast_to(lane, (CHUNK, LANES))
    for s in range(1, NSLICE):
        d_s = ((znorm - scores2[:, s * LANES:(s + 1) * LANES])
               + enorm[:, s * LANES:(s + 1) * LANES])
        lt = d_s < m
        m = jnp.where(lt, d_s, m)
        c = jnp.where(lt, lane + float(s * LANES), c)
    mt = m.T
    ct = c.T
    b = jnp.min(mt, axis=0, keepdims=True)
    cand = jnp.where(mt == b, ct, float(K))
    res = jnp.min(cand, axis=0).astype(jnp.int32)
    out_ref[r, pl.ds(h * CHUNK, CHUNK)] = res


@jax.jit
def kernel(z_e_x, codebook):
    b, t, d = z_e_x.shape
    m = b * t
    nchunks = m // CHUNK
    cb2 = 2.0 * codebook
    flat = z_e_x.reshape(m, d)
    znl = jnp.sum(flat ** 2, axis=1).reshape(nchunks, 1, CHUNK)
    enorm = jnp.sum(codebook ** 2, axis=1)[None, :]
    out = pl.pallas_call(
        _vq_kernel,
        grid=(nchunks,),
        in_specs=[
            pl.BlockSpec((ROWS_PER_BLOCK, t, d),
                         lambda i: (i // CHUNKS_PER_BLOCK, 0, 0)),
            pl.BlockSpec((K, d), lambda i: (0, 0)),
            pl.BlockSpec((1, 1, CHUNK), lambda i: (i, 0, 0)),
            pl.BlockSpec((1, K), lambda i: (0, 0)),
        ],
        out_specs=pl.BlockSpec((ROWS_PER_BLOCK, t),
                               lambda i: (i // CHUNKS_PER_BLOCK, 0)),
        out_shape=jax.ShapeDtypeStruct((b, t), jnp.int32),
    )(z_e_x, cb2, znl, enorm)
    return out


# speedup vs baseline: 2.2408x; 1.2574x over previous
import jax
import jax.numpy as jnp
from jax.experimental import pallas as pl
from jax.experimental.pallas import tpu as pltpu

K = 1024
D = 64
LANES = 128
NSLICE = K // LANES
T_FIXED = 1024
ROWS_PER_CHUNK = 2
CHUNK = ROWS_PER_CHUNK * T_FIXED
ROWS_PER_BLOCK = 8
CHUNKS_PER_BLOCK = ROWS_PER_BLOCK // ROWS_PER_CHUNK
NCHUNKS = 32 // ROWS_PER_CHUNK


def _vq_kernel(x_ref, cb_ref, out_ref, cb2_ref, en_ref):
    i = pl.program_id(0)
    rm = (i % CHUNKS_PER_BLOCK) * ROWS_PER_CHUNK

    @pl.when(i == 0)
    def _init():
        cb = cb_ref[...]
        cb2_ref[...] = cb + cb
        en_col = jnp.sum(cb ** 2, axis=1, keepdims=True)
        en_ref[...] = en_col.T

    x = x_ref[pl.ds(rm, ROWS_PER_CHUNK), :, :].reshape(CHUNK, D)
    znorm = jnp.sum(x ** 2, axis=1, keepdims=True)
    enorm = en_ref[...]
    scores2 = jax.lax.dot_general(
        x, cb2_ref[...],
        dimension_numbers=(((1,), (1,)), ((), ())),
        preferred_element_type=jnp.float32,
    )
    lane = jax.lax.broadcasted_iota(jnp.int32, (1, LANES), 1).astype(jnp.float32)
    m = (znorm - scores2[:, 0:LANES]) + enorm[:, 0:LANES]
    c = jnp.broadcast_to(lane, (CHUNK, LANES))
    for s in range(1, NSLICE):
        d_s = ((znorm - scores2[:, s * LANES:(s + 1) * LANES])
               + enorm[:, s * LANES:(s + 1) * LANES])
        lt = d_s < m
        m = jnp.where(lt, d_s, m)
        c = jnp.where(lt, lane + float(s * LANES), c)
    mt = m.T
    ct = c.T
    b = jnp.min(mt, axis=0, keepdims=True)
    cand = jnp.where(mt == b, ct, float(K))
    res = jnp.min(cand, axis=0).astype(jnp.int32)
    for q in range(ROWS_PER_CHUNK):
        out_ref[rm + q, :] = res[q * T_FIXED:(q + 1) * T_FIXED]


@jax.jit
def kernel(z_e_x, codebook):
    b, t, d = z_e_x.shape
    out = pl.pallas_call(
        _vq_kernel,
        grid=(NCHUNKS,),
        in_specs=[
            pl.BlockSpec((ROWS_PER_BLOCK, t, d),
                         lambda i: (i // CHUNKS_PER_BLOCK, 0, 0)),
            pl.BlockSpec((K, d), lambda i: (0, 0)),
        ],
        out_specs=pl.BlockSpec((ROWS_PER_BLOCK, t),
                               lambda i: (i // CHUNKS_PER_BLOCK, 0)),
        out_shape=jax.ShapeDtypeStruct((b, t), jnp.int32),
        scratch_shapes=[
            pltpu.VMEM((K, d), jnp.float32),
            pltpu.VMEM((1, K), jnp.float32),
        ],
    )(z_e_x, codebook)
    return out


# speedup vs baseline: 2.3374x; 1.0431x over previous
import jax
import jax.numpy as jnp
from jax.experimental import pallas as pl
from jax.experimental.pallas import tpu as pltpu

K = 1024
D = 64
LANES = 128
NSLICE = K // LANES
T_FIXED = 1024
ROWS_PER_CHUNK = 4
CHUNK = ROWS_PER_CHUNK * T_FIXED
ROWS_PER_BLOCK = 8
CHUNKS_PER_BLOCK = ROWS_PER_BLOCK // ROWS_PER_CHUNK
NCHUNKS = 32 // ROWS_PER_CHUNK


def _vq_kernel(x_ref, cb_ref, out_ref, cb2_ref, en_ref):
    i = pl.program_id(0)
    rm = (i % CHUNKS_PER_BLOCK) * ROWS_PER_CHUNK

    @pl.when(i == 0)
    def _init():
        cb = cb_ref[...]
        cb2_ref[...] = cb + cb
        en_col = jnp.sum(cb ** 2, axis=1, keepdims=True)
        en_ref[...] = en_col.T

    x = x_ref[pl.ds(rm, ROWS_PER_CHUNK), :, :].reshape(CHUNK, D)
    znorm = jnp.sum(x ** 2, axis=1, keepdims=True)
    enorm = en_ref[...]
    scores2 = jax.lax.dot_general(
        x, cb2_ref[...],
        dimension_numbers=(((1,), (1,)), ((), ())),
        preferred_element_type=jnp.float32,
    )
    lane = jax.lax.broadcasted_iota(jnp.int32, (1, LANES), 1).astype(jnp.float32)
    m = (znorm - scores2[:, 0:LANES]) + enorm[:, 0:LANES]
    c = jnp.broadcast_to(lane, (CHUNK, LANES))
    for s in range(1, NSLICE):
        d_s = ((znorm - scores2[:, s * LANES:(s + 1) * LANES])
               + enorm[:, s * LANES:(s + 1) * LANES])
        lt = d_s < m
        m = jnp.where(lt, d_s, m)
        c = jnp.where(lt, lane + float(s * LANES), c)
    mt = m.T
    ct = c.T
    b = jnp.min(mt, axis=0, keepdims=True)
    cand = jnp.where(mt == b, ct, float(K))
    res = jnp.min(cand, axis=0).astype(jnp.int32)
    for q in range(ROWS_PER_CHUNK):
        out_ref[rm + q, :] = res[q * T_FIXED:(q + 1) * T_FIXED]


@jax.jit
def kernel(z_e_x, codebook):
    b, t, d = z_e_x.shape
    out = pl.pallas_call(
        _vq_kernel,
        grid=(NCHUNKS,),
        in_specs=[
            pl.BlockSpec((ROWS_PER_BLOCK, t, d),
                         lambda i: (i // CHUNKS_PER_BLOCK, 0, 0)),
            pl.BlockSpec((K, d), lambda i: (0, 0)),
        ],
        out_specs=pl.BlockSpec((ROWS_PER_BLOCK, t),
                               lambda i: (i // CHUNKS_PER_BLOCK, 0)),
        out_shape=jax.ShapeDtypeStruct((b, t), jnp.int32),
        scratch_shapes=[
            pltpu.VMEM((K, d), jnp.float32),
            pltpu.VMEM((1, K), jnp.float32),
        ],
    )(z_e_x, codebook)
    return out


# speedup vs baseline: 2.3880x; 1.0217x over previous
import jax
import jax.numpy as jnp
from jax.experimental import pallas as pl
from jax.experimental.pallas import tpu as pltpu

K = 1024
D = 64
LANES = 128
NSLICE = K // LANES
T_FIXED = 1024
ROWS_PER_CHUNK = 8
CHUNK = ROWS_PER_CHUNK * T_FIXED
ROWS_PER_BLOCK = 8
CHUNKS_PER_BLOCK = ROWS_PER_BLOCK // ROWS_PER_CHUNK
NCHUNKS = 32 // ROWS_PER_CHUNK


def _vq_kernel(x_ref, cb_ref, out_ref, cb2_ref, en_ref):
    i = pl.program_id(0)
    rm = (i % CHUNKS_PER_BLOCK) * ROWS_PER_CHUNK

    @pl.when(i == 0)
    def _init():
        cb = cb_ref[...]
        cb2_ref[...] = cb + cb
        en_col = jnp.sum(cb ** 2, axis=1, keepdims=True)
        en_ref[...] = en_col.T

    x = x_ref[pl.ds(rm, ROWS_PER_CHUNK), :, :].reshape(CHUNK, D)
    znorm = jnp.sum(x ** 2, axis=1, keepdims=True)
    enorm = en_ref[...]
    scores2 = jax.lax.dot_general(
        x, cb2_ref[...],
        dimension_numbers=(((1,), (1,)), ((), ())),
        preferred_element_type=jnp.float32,
    )
    lane = jax.lax.broadcasted_iota(jnp.int32, (1, LANES), 1).astype(jnp.float32)
    m = (znorm - scores2[:, 0:LANES]) + enorm[:, 0:LANES]
    c = jnp.broadcast_to(lane, (CHUNK, LANES))
    for s in range(1, NSLICE):
        d_s = ((znorm - scores2[:, s * LANES:(s + 1) * LANES])
               + enorm[:, s * LANES:(s + 1) * LANES])
        lt = d_s < m
        m = jnp.where(lt, d_s, m)
        c = jnp.where(lt, lane + float(s * LANES), c)
    mt = m.T
    ct = c.T
    b = jnp.min(mt, axis=0, keepdims=True)
    cand = jnp.where(mt == b, ct, float(K))
    res = jnp.min(cand, axis=0).astype(jnp.int32)
    for q in range(ROWS_PER_CHUNK):
        out_ref[rm + q, :] = res[q * T_FIXED:(q + 1) * T_FIXED]


@jax.jit
def kernel(z_e_x, codebook):
    b, t, d = z_e_x.shape
    out = pl.pallas_call(
        _vq_kernel,
        grid=(NCHUNKS,),
        in_specs=[
            pl.BlockSpec((ROWS_PER_BLOCK, t, d),
                         lambda i: (i // CHUNKS_PER_BLOCK, 0, 0)),
            pl.BlockSpec((K, d), lambda i: (0, 0)),
        ],
        out_specs=pl.BlockSpec((ROWS_PER_BLOCK, t),
                               lambda i: (i // CHUNKS_PER_BLOCK, 0)),
        out_shape=jax.ShapeDtypeStruct((b, t), jnp.int32),
        scratch_shapes=[
            pltpu.VMEM((K, d), jnp.float32),
            pltpu.VMEM((1, K), jnp.float32),
        ],
    )(z_e_x, codebook)
    return out
